# baseline (speedup 1.0000x reference)
import functools

import jax
import jax.numpy as jnp
from jax import lax
from jax.experimental import pallas as pl
from jax.experimental.pallas import tpu as pltpu
from jax.experimental.pallas import tpu_sc as plsc

K = 10
NUM_LEVELS = 2 ** K
BASE_SLICE = 2.0 ** (-K)
INV_SLICE = float(2.0 ** K)
N = 4194304

NUM_CORES = 2
NUM_SUBCORES = 16
NW = NUM_CORES * NUM_SUBCORES
PER_WORKER = N // NW
CHUNK = 16384
NCHUNK = PER_WORKER // CHUNK
NBUF = 6
LANES = 16
GROUPS = CHUNK // LANES
UNROLL = 8


def _quantize16(v):
    y = jnp.minimum(jnp.maximum(v * INV_SLICE, jnp.float32(0.0)),
                    jnp.float32(NUM_LEVELS - 1))
    return y.astype(jnp.int32).astype(jnp.float32) * jnp.float32(BASE_SLICE)


def _quantize_chunk(buf):
    def body(i, carry):
        base = i * (LANES * UNROLL)
        for j in range(UNROLL):
            sl = pl.ds(base + j * LANES, LANES)
            buf[sl] = _quantize16(buf[sl])
        return carry
    lax.fori_loop(0, GROUPS // UNROLL, body, 0)


@functools.partial(
    pl.kernel,
    mesh=plsc.VectorSubcoreMesh(core_axis_name="c", subcore_axis_name="s"),
    out_type=jax.ShapeDtypeStruct((N,), jnp.float32),
    scratch_types=(
        [pltpu.VMEM((CHUNK,), jnp.float32)] * NBUF
        + [pltpu.SemaphoreType.DMA] * (2 * NBUF)
    ),
)
def _sc_encode(x_hbm, out_hbm, *scratch):
    wid = lax.axis_index("s") * NUM_CORES + lax.axis_index("c")
    base = wid * PER_WORKER
    bufs = scratch[:NBUF]
    in_sems = scratch[NBUF:2 * NBUF]
    out_sems = scratch[2 * NBUF:]
    in_copies = [None] * NBUF
    out_copies = [None] * NBUF

    for k in range(min(NBUF - 1, NCHUNK)):
        in_copies[k] = pltpu.async_copy(
            x_hbm.at[pl.ds(base + k * CHUNK, CHUNK)], bufs[k], in_sems[k])
    for k in range(NCHUNK):
        cur = k % NBUF
        in_copies[cur].wait()
        _quantize_chunk(bufs[cur])
        out_copies[cur] = pltpu.async_copy(
            bufs[cur], out_hbm.at[pl.ds(base + k * CHUNK, CHUNK)],
            out_sems[cur])
        pre = k + NBUF - 1
        if pre < NCHUNK:
            pb = pre % NBUF
            if out_copies[pb] is not None:
                out_copies[pb].wait()
            in_copies[pb] = pltpu.async_copy(
                x_hbm.at[pl.ds(base + pre * CHUNK, CHUNK)],
                bufs[pb], in_sems[pb])
    for b in range(NBUF):
        if out_copies[b] is not None and b != (NCHUNK - 1) % NBUF:
            out_copies[b].wait()
    out_copies[(NCHUNK - 1) % NBUF].wait()


def kernel(x):
    return _sc_encode(x)


# speedup vs baseline: 1.0340x; 1.0340x over previous
import functools

import jax
import jax.numpy as jnp
from jax import lax
from jax.experimental import pallas as pl
from jax.experimental.pallas import tpu as pltpu
from jax.experimental.pallas import tpu_sc as plsc

K = 10
NUM_LEVELS = 2 ** K
BASE_SLICE = 2.0 ** (-K)
INV_SLICE = float(2.0 ** K)
N = 4194304

NUM_CORES = 2
NUM_SUBCORES = 16
NW = NUM_CORES * NUM_SUBCORES
PER_WORKER = N // NW
CHUNK = 8192
NCHUNK = PER_WORKER // CHUNK
NBUF = 4
LANES = 16
GROUPS = CHUNK // LANES
UNROLL = 8


def _quantize16(v):
    y = jnp.minimum(jnp.maximum(v * INV_SLICE, jnp.float32(0.0)),
                    jnp.float32(NUM_LEVELS - 1))
    return y.astype(jnp.int32).astype(jnp.float32) * jnp.float32(BASE_SLICE)


def _quantize_chunk(buf):
    def body(i, carry):
        base = i * (LANES * UNROLL)
        for j in range(UNROLL):
            sl = pl.ds(base + j * LANES, LANES)
            buf[sl] = _quantize16(buf[sl])
        return carry
    lax.fori_loop(0, GROUPS // UNROLL, body, 0)


@functools.partial(
    pl.kernel,
    mesh=plsc.VectorSubcoreMesh(core_axis_name="c", subcore_axis_name="s"),
    out_type=jax.ShapeDtypeStruct((N,), jnp.float32),
    scratch_types=(
        [pltpu.VMEM((CHUNK,), jnp.float32)] * NBUF
        + [pltpu.SemaphoreType.DMA] * (2 * NBUF)
    ),
)
def _sc_encode(x_hbm, out_hbm, *scratch):
    wid = lax.axis_index("s") * NUM_CORES + lax.axis_index("c")
    base = wid * PER_WORKER
    bufs = scratch[:NBUF]
    in_sems = scratch[NBUF:2 * NBUF]
    out_sems = scratch[2 * NBUF:]
    in_copies = [None] * NBUF
    out_copies = [None] * NBUF

    for k in range(min(NBUF - 1, NCHUNK)):
        in_copies[k] = pltpu.async_copy(
            x_hbm.at[pl.ds(base + k * CHUNK, CHUNK)], bufs[k], in_sems[k])
    for k in range(NCHUNK):
        cur = k % NBUF
        in_copies[cur].wait()
        _quantize_chunk(bufs[cur])
        out_copies[cur] = pltpu.async_copy(
            bufs[cur], out_hbm.at[pl.ds(base + k * CHUNK, CHUNK)],
            out_sems[cur])
        pre = k + NBUF - 1
        if pre < NCHUNK:
            pb = pre % NBUF
            if out_copies[pb] is not None:
                out_copies[pb].wait()
            in_copies[pb] = pltpu.async_copy(
                x_hbm.at[pl.ds(base + pre * CHUNK, CHUNK)],
                bufs[pb], in_sems[pb])
    for b in range(NBUF):
        if out_copies[b] is not None and b != (NCHUNK - 1) % NBUF:
            out_copies[b].wait()
    out_copies[(NCHUNK - 1) % NBUF].wait()


def kernel(x):
    return _sc_encode(x)
